# Initial kernel scaffold; baseline (speedup 1.0000x reference)
import jax
import jax.numpy as jnp
from jax.experimental import pallas as pl


def kernel(history_indices, history_ratings, movie_embeddings):
    raise NotImplementedError("write your pallas kernel here")


# speedup vs baseline: 2.8504x; 2.8504x over previous
import functools

import jax
import jax.numpy as jnp
from jax import lax
from jax.experimental import pallas as pl
from jax.experimental.pallas import tpu as pltpu
from jax.experimental.pallas import tpu_sc as plsc

NUM_CORES = 2
NUM_SUBCORES = 16
NUM_WORKERS = NUM_CORES * NUM_SUBCORES
LANES = 16

B = 16384
L = 50
LPAD = 64
D = 32
BPW = B // NUM_WORKERS
NBUF = 8

_BCAST_DNUMS = lax.GatherDimensionNumbers(
    offset_dims=(), collapsed_slice_dims=(0,), start_index_map=(0,))


def _bcast_lane(vec, lane):
    idx = jnp.full((LANES, 1), lane, dtype=jnp.int32)
    return lax.gather(vec, idx, _BCAST_DNUMS, (1,),
                      mode=lax.GatherScatterMode.PROMISE_IN_BOUNDS)


def _body(idx_hbm, rat_hbm, table_hbm, out_hbm, idx_v, rat_v, rows_v, out_v,
          *sems):
    wid = lax.axis_index("s") * NUM_CORES + lax.axis_index("c")
    base = wid * BPW

    pltpu.sync_copy(idx_hbm.at[pl.ds(base, BPW)], idx_v)
    pltpu.sync_copy(rat_hbm.at[pl.ds(base, BPW)], rat_v)

    for b in range(NBUF):
        pltpu.async_copy(table_hbm.at[idx_v.at[b]], rows_v.at[b], sems[b])

    def block(blk, carry):
        u0 = blk * NBUF
        for b in range(NBUF):
            u = u0 + b
            pltpu.make_async_copy(
                table_hbm.at[idx_v.at[u]], rows_v.at[b], sems[b]).wait()

            rvecs = [rat_v[u, pl.ds(16 * j, 16)] for j in range(4)]
            acc0 = jnp.zeros((LANES,), jnp.float32)
            acc1 = jnp.zeros((LANES,), jnp.float32)
            for l in range(L):
                r = _bcast_lane(rvecs[l // 16], l % 16)
                acc0 = acc0 + rows_v[b, l, pl.ds(0, 16)] * r
                acc1 = acc1 + rows_v[b, l, pl.ds(16, 16)] * r
            out_v[u, pl.ds(0, 16)] = acc0
            out_v[u, pl.ds(16, 16)] = acc1

            @pl.when(u0 < BPW - NBUF)
            def _():
                pltpu.async_copy(
                    table_hbm.at[idx_v.at[u + NBUF]], rows_v.at[b], sems[b])

        return carry

    lax.fori_loop(0, BPW // NBUF, block, 0)

    pltpu.sync_copy(out_v, out_hbm.at[pl.ds(base, BPW)])


@jax.jit
def _encode(history_indices, ratings_padded, movie_embeddings):
    mesh = plsc.VectorSubcoreMesh(
        core_axis_name="c", subcore_axis_name="s", num_cores=NUM_CORES,
        num_subcores=NUM_SUBCORES)
    return pl.kernel(
        _body,
        out_type=jax.ShapeDtypeStruct((B, D), jnp.float32),
        mesh=mesh,
        scratch_types=[
            pltpu.VMEM((BPW, L), jnp.int32),
            pltpu.VMEM((BPW, LPAD), jnp.float32),
            pltpu.VMEM((NBUF, L, D), jnp.float32),
            pltpu.VMEM((BPW, D), jnp.float32),
        ] + [pltpu.SemaphoreType.DMA] * NBUF,
        compiler_params=pltpu.CompilerParams(use_tc_tiling_on_sc=False),
    )(history_indices, ratings_padded, movie_embeddings)


def kernel(history_indices, history_ratings, movie_embeddings):
    ratings_padded = jnp.pad(history_ratings, ((0, 0), (0, LPAD - L)))
    return _encode(history_indices, ratings_padded, movie_embeddings)


# speedup vs baseline: 4.6636x; 1.6362x over previous
import jax
import jax.numpy as jnp
from jax import lax
from jax.experimental import pallas as pl
from jax.experimental.pallas import tpu as pltpu
from jax.experimental.pallas import tpu_sc as plsc

NUM_CORES = 2
NUM_SUBCORES = 16
NUM_WORKERS = NUM_CORES * NUM_SUBCORES
LANES = 16

NUM_ROWS = 1000000
B = 16384
L = 50
LPAD = 64
D = 32
BPW = B // NUM_WORKERS
NBUF = 8

W = 32768
QUARTER = W // 4
QSHIFT = QUARTER.bit_length() - 1
FULL_BLOCKS = NUM_ROWS // W
TAIL_START = FULL_BLOCKS * W
TAIL_ROWS = NUM_ROWS - TAIL_START
OUT_ROWS = NUM_ROWS * D // 128

_BCAST_DNUMS = lax.GatherDimensionNumbers(
    offset_dims=(), collapsed_slice_dims=(0,), start_index_map=(0,))


def _bcast_lane(vec, lane):
    idx = jnp.full((LANES, 1), lane, dtype=jnp.int32)
    return lax.gather(vec, idx, _BCAST_DNUMS, (1,),
                      mode=lax.GatherScatterMode.PROMISE_IN_BOUNDS)


def _transpose_block(x_ref, tail_ref, o_ref):
    blk = pl.program_id(0)

    @pl.when(blk < FULL_BLOCKS)
    def _():
        y = x_ref[...].T
        o_ref[...] = jnp.concatenate(
            [y[q * QUARTER:(q + 1) * QUARTER, :] for q in range(4)], axis=1)

    @pl.when(blk == FULL_BLOCKS)
    def _():
        o_ref[0:TAIL_ROWS * D // 128, :] = tail_ref[...]


W2 = 8192
LG = 56


def _pack_inputs(idx_ref, rat_ref, oi_ref, or_ref):
    i = idx_ref[...]
    q = (i & ~(W - 1)) | ((i & (QUARTER - 1)) << 2) | ((i >> QSHIFT) & 3)
    yi = jnp.where(i >= TAIL_START, i, q).T
    yr = rat_ref[...].T

    pad_i = lax.broadcasted_iota(jnp.int32, (W2, LPAD - L), 0)
    pad_r = jnp.zeros((W2, LPAD - L), jnp.float32)

    def pack(y, pad):
        yp = jnp.concatenate([y, pad], axis=1)
        return jnp.concatenate([yp[:W2 // 2], yp[W2 // 2:]], axis=1)

    oi_ref[...] = pack(yi, pad_i)
    or_ref[...] = pack(yr, pad_r)


def _encode_body(idx_hbm, rat_hbm, table_hbm, out_hbm, idx_v, rat_v, rows_v,
                 out_v, *sems):
    w = lax.axis_index("s") * NUM_CORES + lax.axis_index("c")
    rows = (w // 16) * (W2 // 2) + (w % 8) * BPW
    col = ((w % 16) // 8) * LPAD

    pltpu.sync_copy(idx_hbm.at[pl.ds(rows, BPW), pl.ds(col, LPAD)], idx_v)
    pltpu.sync_copy(rat_hbm.at[pl.ds(rows, BPW), pl.ds(col, LPAD)], rat_v)

    def idx_ref(u):
        return idx_v.at[u, pl.ds(0, LG)]

    for b in range(NBUF):
        pltpu.async_copy(table_hbm.at[idx_ref(b)], rows_v.at[b], sems[b])

    def block(blk, carry):
        u0 = blk * NBUF
        for b in range(NBUF):
            u = u0 + b
            pltpu.make_async_copy(
                table_hbm.at[idx_ref(u)], rows_v.at[b], sems[b]).wait()

            rvecs = [rat_v[u, pl.ds(16 * j, 16)] for j in range(4)]
            acc0 = jnp.zeros((LANES,), jnp.float32)
            acc1 = jnp.zeros((LANES,), jnp.float32)
            for l in range(L):
                r = _bcast_lane(rvecs[l // 16], l % 16)
                acc0 = acc0 + rows_v[b, l, pl.ds(0, 16)] * r
                acc1 = acc1 + rows_v[b, l, pl.ds(16, 16)] * r
            out_v[pl.ds(u * D, 16)] = acc0
            out_v[pl.ds(u * D + 16, 16)] = acc1

            @pl.when(u0 < BPW - NBUF)
            def _():
                pltpu.async_copy(
                    table_hbm.at[idx_ref(u + NBUF)], rows_v.at[b], sems[b])

        return carry

    lax.fori_loop(0, BPW // NBUF, block, 0)

    pltpu.sync_copy(out_v, out_hbm.at[pl.ds(w * (BPW * D), BPW * D)])


@jax.jit
def _run(idx_t, rat_t, table_t, table_tail):
    table_packed = pl.pallas_call(
        _transpose_block,
        grid=(FULL_BLOCKS + 1,),
        in_specs=[
            pl.BlockSpec((32, W), lambda j: (0, j)),
            pl.BlockSpec((TAIL_ROWS * D // 128, 128), lambda j: (0, 0)),
        ],
        out_specs=pl.BlockSpec((QUARTER, 128), lambda j: (j, 0)),
        out_shape=jax.ShapeDtypeStruct((OUT_ROWS, 128), jnp.float32),
    )(table_t, table_tail)

    table_lin = table_packed.reshape(NUM_ROWS, D)

    idx_packed, rat_packed = pl.pallas_call(
        _pack_inputs,
        grid=(B // W2,),
        in_specs=[
            pl.BlockSpec((L, W2), lambda j: (0, j)),
            pl.BlockSpec((L, W2), lambda j: (0, j)),
        ],
        out_specs=[
            pl.BlockSpec((W2 // 2, 2 * LPAD), lambda j: (j, 0)),
            pl.BlockSpec((W2 // 2, 2 * LPAD), lambda j: (j, 0)),
        ],
        out_shape=[
            jax.ShapeDtypeStruct((B // 2, 2 * LPAD), jnp.int32),
            jax.ShapeDtypeStruct((B // 2, 2 * LPAD), jnp.float32),
        ],
    )(idx_t, rat_t)

    mesh = plsc.VectorSubcoreMesh(
        core_axis_name="c", subcore_axis_name="s", num_cores=NUM_CORES,
        num_subcores=NUM_SUBCORES)
    out_flat = pl.kernel(
        _encode_body,
        out_type=jax.ShapeDtypeStruct((B * D,), jnp.float32),
        mesh=mesh,
        scratch_types=[
            pltpu.VMEM((BPW, LPAD), jnp.int32),
            pltpu.VMEM((BPW, LPAD), jnp.float32),
            pltpu.VMEM((NBUF, LG, D), jnp.float32),
            pltpu.VMEM((BPW * D,), jnp.float32),
        ] + [pltpu.SemaphoreType.DMA] * NBUF,
        compiler_params=pltpu.CompilerParams(use_tc_tiling_on_sc=False),
    )(idx_packed, rat_packed, table_lin)
    return out_flat.reshape(B, D)


def kernel(history_indices, history_ratings, movie_embeddings):
    idx_t = history_indices.astype(jnp.int32).T
    rat_t = history_ratings.T
    table_t = movie_embeddings.T
    table_tail = movie_embeddings[TAIL_START:, :].reshape(
        TAIL_ROWS * D // 128, 128)
    return _run(idx_t, rat_t, table_t, table_tail)
